# baseline (speedup 1.0000x reference)
import jax
import jax.numpy as jnp
from jax.experimental import pallas as pl
from jax.experimental.pallas import tpu as pltpu


_ROWS = 16384
_COLS = 4096
def _copy_body(in_ref, out_ref, sem):
    copy = pltpu.make_async_copy(in_ref, out_ref, sem)
    copy.start()
    copy.wait()


def kernel(tensor):
    flat = tensor.reshape(_ROWS, _COLS)
    out = pl.pallas_call(
        _copy_body,
        in_specs=[pl.BlockSpec(memory_space=pl.ANY)],
        out_specs=pl.BlockSpec(memory_space=pl.ANY),
        out_shape=jax.ShapeDtypeStruct((_ROWS, _COLS), jnp.float32),
        scratch_shapes=[pltpu.SemaphoreType.DMA],
    )(flat)
    return out.reshape(tensor.shape[0], tensor.shape[1], _COLS)


# speedup vs baseline: 21.3373x; 21.3373x over previous
import jax
import jax.numpy as jnp
from jax import lax
from jax.experimental import pallas as pl
from jax.experimental.pallas import tpu as pltpu
from jax.experimental.pallas import tpu_sc as plsc


_ROWS = 16384
_COLS = 4096
_NC = 2
_NS = 16
_NW = _NC * _NS
_RPW = _ROWS // _NW
_RB = 8
_NBUF = 2
_NCH = _RPW // _RB


def _sc_body(in_hbm, out_hbm, buf, sin, sout):
    wid = lax.axis_index("s") * _NC + lax.axis_index("c")
    base = wid * _RPW

    def _in(c, b):
        row = base + c * _RB
        return pltpu.make_async_copy(
            in_hbm.at[pl.ds(row, _RB), :], buf.at[b], sin.at[b])

    def _out(c, b):
        row = base + c * _RB
        return pltpu.make_async_copy(
            buf.at[b], out_hbm.at[pl.ds(row, _RB), :], sout.at[b])

    for b in range(_NBUF):
        _in(b, b).start()

    def step(it, carry):
        for b in range(_NBUF):
            c = it * _NBUF + b
            _in(c, b).wait()
            _out(c, b).start()
            _out(c, b).wait()
            nc = c + _NBUF

            @pl.when(nc < _NCH)
            def _():
                _in(nc, b).start()
        return carry

    lax.fori_loop(0, _NCH // _NBUF, step, 0)


def kernel(tensor):
    flat = tensor.reshape(_ROWS, _COLS)
    k = pl.kernel(
        _sc_body,
        out_type=jax.ShapeDtypeStruct((_ROWS, _COLS), jnp.float32),
        mesh=plsc.VectorSubcoreMesh(core_axis_name="c", subcore_axis_name="s"),
        scratch_types=[
            pltpu.VMEM((_NBUF, _RB, _COLS), jnp.float32),
            pltpu.SemaphoreType.DMA((_NBUF,)),
            pltpu.SemaphoreType.DMA((_NBUF,)),
        ],
    )
    out = k(flat)
    return out.reshape(tensor.shape[0], tensor.shape[1], _COLS)
